# initial kernel scaffold (unmeasured)
import jax
import jax.numpy as jnp
from jax import lax
from jax.experimental import pallas as pl
from jax.experimental.pallas import tpu as pltpu


def kernel(
    x,
):
    def body(*refs):
        pass

    out_shape = jax.ShapeDtypeStruct(..., jnp.float32)
    return pl.pallas_call(body, out_shape=out_shape)(...)


# baseline (device time: 57376 ns/iter reference)
import jax
import jax.numpy as jnp
from jax import lax
from jax.experimental import pallas as pl
from jax.experimental.pallas import tpu as pltpu

N_DEV = 4


def kernel(x):
    m_rows, n_cols = x.shape

    def body(x_ref, out_ref, comm_ref, send_sems, recv_sems):
        my_pos = lax.axis_index("i")

        xv = x_ref[...]
        m = jnp.max(xv, axis=1, keepdims=True)
        e = jnp.exp(xv - m)
        s = jnp.sum(e, axis=1, keepdims=True)
        comm_ref[my_pos] = jnp.concatenate([m, s], axis=1)

        barrier_sem = pltpu.get_barrier_semaphore()
        for k in range(1, N_DEV):
            pl.semaphore_signal(
                barrier_sem, inc=1,
                device_id=(lax.rem(my_pos + k, N_DEV),),
                device_id_type=pl.DeviceIdType.MESH,
            )
        pl.semaphore_wait(barrier_sem, N_DEV - 1)

        sends = []
        for k in range(1, N_DEV):
            tgt = lax.rem(my_pos + k, N_DEV)
            rdma = pltpu.make_async_remote_copy(
                src_ref=comm_ref.at[my_pos],
                dst_ref=comm_ref.at[my_pos],
                send_sem=send_sems.at[k],
                recv_sem=recv_sems.at[my_pos],
                device_id=(tgt,),
                device_id_type=pl.DeviceIdType.MESH,
            )
            rdma.start()
            sends.append(rdma)

        for k in range(1, N_DEV):
            src_o = lax.rem(my_pos + k, N_DEV)
            recv = pltpu.make_async_remote_copy(
                src_ref=comm_ref.at[my_pos],
                dst_ref=comm_ref.at[src_o],
                send_sem=send_sems.at[0],
                recv_sem=recv_sems.at[src_o],
                device_id=(my_pos,),
                device_id_type=pl.DeviceIdType.MESH,
            )
            recv.wait_recv()
        for rdma in sends:
            rdma.wait_send()

        stats = comm_ref[...]
        m_all = stats[:, :, 0:1]
        s_all = stats[:, :, 1:2]
        g_max = jnp.max(m_all, axis=0)
        g_sum = jnp.sum(s_all * jnp.exp(m_all - g_max[None]), axis=0)
        out_ref[...] = e * (jnp.exp(m - g_max) / g_sum)

    return pl.pallas_call(
        body,
        out_shape=jax.ShapeDtypeStruct((m_rows, n_cols), x.dtype),
        in_specs=[pl.BlockSpec(memory_space=pltpu.VMEM)],
        out_specs=pl.BlockSpec(memory_space=pltpu.VMEM),
        scratch_shapes=[
            pltpu.VMEM((N_DEV, m_rows, 2), jnp.float32),
            pltpu.SemaphoreType.DMA((N_DEV,)),
            pltpu.SemaphoreType.DMA((N_DEV,)),
        ],
        compiler_params=pltpu.CompilerParams(
            collective_id=0,
            vmem_limit_bytes=100 * 1024 * 1024,
        ),
    )(x)


# device time: 34187 ns/iter; 1.6783x vs baseline; 1.6783x over previous
import jax
import jax.numpy as jnp
from jax import lax
from jax.experimental import pallas as pl
from jax.experimental.pallas import tpu as pltpu

N_DEV = 4


def kernel(x):
    m_rows, n_cols = x.shape

    def body(x_ref, out_ref, comm_ref, send_sems, recv_sems):
        my_pos = lax.axis_index("i")

        xv = x_ref[...]
        m = jnp.max(xv, axis=1, keepdims=True)
        e = jnp.exp(xv - m)
        s = jnp.sum(e, axis=1, keepdims=True)
        rows = m_rows // 128
        comm_ref[my_pos] = jnp.concatenate(
            [m.reshape(rows, 128), s.reshape(rows, 128)], axis=0
        )

        barrier_sem = pltpu.get_barrier_semaphore()
        for k in range(1, N_DEV):
            pl.semaphore_signal(
                barrier_sem, inc=1,
                device_id=(lax.rem(my_pos + k, N_DEV),),
                device_id_type=pl.DeviceIdType.MESH,
            )
        pl.semaphore_wait(barrier_sem, N_DEV - 1)

        sends = []
        for k in range(1, N_DEV):
            tgt = lax.rem(my_pos + k, N_DEV)
            rdma = pltpu.make_async_remote_copy(
                src_ref=comm_ref.at[my_pos],
                dst_ref=comm_ref.at[my_pos],
                send_sem=send_sems.at[k],
                recv_sem=recv_sems.at[my_pos],
                device_id=(tgt,),
                device_id_type=pl.DeviceIdType.MESH,
            )
            rdma.start()
            sends.append(rdma)

        for k in range(1, N_DEV):
            src_o = lax.rem(my_pos + k, N_DEV)
            recv = pltpu.make_async_remote_copy(
                src_ref=comm_ref.at[my_pos],
                dst_ref=comm_ref.at[src_o],
                send_sem=send_sems.at[0],
                recv_sem=recv_sems.at[src_o],
                device_id=(my_pos,),
                device_id_type=pl.DeviceIdType.MESH,
            )
            recv.wait_recv()
        for rdma in sends:
            rdma.wait_send()

        stats = comm_ref[...]
        m_all = stats[:, :rows, :]
        s_all = stats[:, rows:, :]
        g_max = jnp.max(m_all, axis=0)
        g_sum = jnp.sum(s_all * jnp.exp(m_all - g_max[None]), axis=0)
        f = jnp.exp(m.reshape(rows, 128) - g_max) / g_sum
        out_ref[...] = e * f.reshape(m_rows, 1)

    return pl.pallas_call(
        body,
        out_shape=jax.ShapeDtypeStruct((m_rows, n_cols), x.dtype),
        in_specs=[pl.BlockSpec(memory_space=pltpu.VMEM)],
        out_specs=pl.BlockSpec(memory_space=pltpu.VMEM),
        scratch_shapes=[
            pltpu.VMEM((N_DEV, 2 * (m_rows // 128), 128), jnp.float32),
            pltpu.SemaphoreType.DMA((N_DEV,)),
            pltpu.SemaphoreType.DMA((N_DEV,)),
        ],
        compiler_params=pltpu.CompilerParams(
            collective_id=0,
            vmem_limit_bytes=100 * 1024 * 1024,
        ),
    )(x)


# device time: 33537 ns/iter; 1.7108x vs baseline; 1.0194x over previous
import jax
import jax.numpy as jnp
from jax import lax
from jax.experimental import pallas as pl
from jax.experimental.pallas import tpu as pltpu

N_DEV = 4
BLK = 256


def kernel(x):
    m_rows, n_cols = x.shape
    nb = m_rows // BLK
    srows = m_rows // 128

    def body(x_hbm, out_hbm, xbuf, ebuf, comm_ref, load_sems, store_sems,
             send_sems, recv_sems):
        my_pos = lax.axis_index("i")

        barrier_sem = pltpu.get_barrier_semaphore()
        for k in range(1, N_DEV):
            pl.semaphore_signal(
                barrier_sem, inc=1,
                device_id=(lax.rem(my_pos + k, N_DEV),),
                device_id_type=pl.DeviceIdType.MESH,
            )

        def load(b, slot):
            return pltpu.make_async_copy(
                x_hbm.at[pl.ds(b * BLK, BLK), :], xbuf.at[slot],
                load_sems.at[slot],
            )

        load(0, 0).start()

        m_parts, s_parts = [], []
        for b in range(nb):
            if b + 1 < nb:
                load(b + 1, (b + 1) % 2).start()
            load(b, b % 2).wait()
            xv = xbuf[b % 2]
            mb = jnp.max(xv, axis=1, keepdims=True)
            eb = jnp.exp(xv - mb)
            sb = jnp.sum(eb, axis=1, keepdims=True)
            ebuf[pl.ds(b * BLK, BLK), :] = eb
            m_parts.append(mb)
            s_parts.append(sb)

        m = jnp.concatenate(m_parts, axis=0)
        s = jnp.concatenate(s_parts, axis=0)
        comm_ref[my_pos] = jnp.concatenate(
            [m.reshape(srows, 128), s.reshape(srows, 128)], axis=0
        )

        pl.semaphore_wait(barrier_sem, N_DEV - 1)

        sends = []
        for k in range(1, N_DEV):
            tgt = lax.rem(my_pos + k, N_DEV)
            rdma = pltpu.make_async_remote_copy(
                src_ref=comm_ref.at[my_pos],
                dst_ref=comm_ref.at[my_pos],
                send_sem=send_sems.at[k],
                recv_sem=recv_sems.at[my_pos],
                device_id=(tgt,),
                device_id_type=pl.DeviceIdType.MESH,
            )
            rdma.start()
            sends.append(rdma)

        for k in range(1, N_DEV):
            src_o = lax.rem(my_pos + k, N_DEV)
            recv = pltpu.make_async_remote_copy(
                src_ref=comm_ref.at[my_pos],
                dst_ref=comm_ref.at[src_o],
                send_sem=send_sems.at[0],
                recv_sem=recv_sems.at[src_o],
                device_id=(my_pos,),
                device_id_type=pl.DeviceIdType.MESH,
            )
            recv.wait_recv()
        for rdma in sends:
            rdma.wait_send()

        stats = comm_ref[...]
        m_all = stats[:, :srows, :]
        s_all = stats[:, srows:, :]
        g_max = jnp.max(m_all, axis=0)
        g_sum = jnp.sum(s_all * jnp.exp(m_all - g_max[None]), axis=0)
        f = (jnp.exp(m.reshape(srows, 128) - g_max) / g_sum).reshape(m_rows, 1)

        for b in range(nb):
            rs = pl.ds(b * BLK, BLK)
            ebuf[rs, :] = ebuf[rs, :] * f[b * BLK:(b + 1) * BLK, :]
            pltpu.make_async_copy(
                ebuf.at[rs, :], out_hbm.at[rs, :], store_sems.at[b]
            ).start()
        for b in range(nb):
            pltpu.make_async_copy(
                ebuf.at[pl.ds(b * BLK, BLK), :],
                out_hbm.at[pl.ds(b * BLK, BLK), :],
                store_sems.at[b],
            ).wait()

    return pl.pallas_call(
        body,
        out_shape=jax.ShapeDtypeStruct((m_rows, n_cols), x.dtype),
        in_specs=[pl.BlockSpec(memory_space=pl.ANY)],
        out_specs=pl.BlockSpec(memory_space=pl.ANY),
        scratch_shapes=[
            pltpu.VMEM((2, BLK, n_cols), jnp.float32),
            pltpu.VMEM((m_rows, n_cols), jnp.float32),
            pltpu.VMEM((N_DEV, 2 * (m_rows // 128), 128), jnp.float32),
            pltpu.SemaphoreType.DMA((2,)),
            pltpu.SemaphoreType.DMA((m_rows // BLK,)),
            pltpu.SemaphoreType.DMA((N_DEV,)),
            pltpu.SemaphoreType.DMA((N_DEV,)),
        ],
        compiler_params=pltpu.CompilerParams(
            collective_id=0,
            vmem_limit_bytes=100 * 1024 * 1024,
        ),
    )(x)
